# baseline (speedup 1.0000x reference)
import functools

import jax
import jax.numpy as jnp
from jax import lax
from jax.experimental import pallas as pl
from jax.experimental.pallas import tpu as pltpu
from jax.experimental.pallas import tpu_sc as plsc


def _gather_codes(codes, idx):
    n = idx.shape[0]
    _, m = codes.shape
    dt = codes.dtype
    info = plsc.get_sparse_core_info()
    nc, ns = info.num_cores, info.num_subcores
    nw = nc * ns
    n_per_w = n // nw
    ch = 1600
    nch = n_per_w // ch

    mesh = plsc.VectorSubcoreMesh(core_axis_name="c", subcore_axis_name="s")

    def body(codes_hbm, idx_hbm, out_hbm, idx_v, rows_v, sem):
        wid = lax.axis_index("s") * nc + lax.axis_index("c")
        base = wid * n_per_w

        def step(i, carry):
            off = base + i * ch
            pltpu.sync_copy(idx_hbm.at[pl.ds(off, ch)], idx_v)
            pltpu.async_copy(codes_hbm.at[idx_v], rows_v, sem).wait()
            pltpu.sync_copy(rows_v, out_hbm.at[pl.ds(off, ch)])
            return carry

        lax.fori_loop(0, nch, step, 0)

    f = pl.kernel(
        body,
        mesh=mesh,
        out_type=jax.ShapeDtypeStruct((n, m), dt),
        scratch_types=[
            pltpu.VMEM((ch,), jnp.int32),
            pltpu.VMEM((ch, m), dt),
            pltpu.SemaphoreType.DMA,
        ],
        compiler_params=pltpu.CompilerParams(use_tc_tiling_on_sc=False),
    )
    return f(codes, idx)


def _gather_codes_tct(codes_p, idx):
    n = idx.shape[0]
    _, w = codes_p.shape
    info = plsc.get_sparse_core_info()
    nc, ns = info.num_cores, info.num_subcores
    nw = nc * ns
    n_per_w = n // nw
    ch = 800
    nch = n_per_w // ch

    mesh = plsc.VectorSubcoreMesh(core_axis_name="c", subcore_axis_name="s")

    def body(codes_hbm, idx_hbm, out_hbm, idx_v, rows_v, sem):
        wid = lax.axis_index("s") * nc + lax.axis_index("c")
        base = wid * n_per_w

        def step(i, carry):
            off = base + i * ch
            pltpu.sync_copy(idx_hbm.at[pl.ds(off, ch)], idx_v)
            pltpu.async_copy(codes_hbm.at[idx_v], rows_v, sem).wait()
            pltpu.sync_copy(rows_v, out_hbm.at[pl.ds(off, ch)])
            return carry

        lax.fori_loop(0, nch, step, 0)

    f = pl.kernel(
        body,
        mesh=mesh,
        out_type=jax.ShapeDtypeStruct((n, w), jnp.int32),
        scratch_types=[
            pltpu.VMEM((ch,), jnp.int32),
            pltpu.VMEM((ch, w), jnp.int32),
            pltpu.SemaphoreType.DMA,
        ],
        compiler_params=pltpu.CompilerParams(use_tc_tiling_on_sc=True),
    )
    return f(codes_p, idx)


def _combine(wc, cbt, t=2048, interpret=False):
    n, w = wc.shape
    m, d, k = cbt.shape
    grid = n // t

    def body(wc_ref, cbt_ref, out_ref):
        one = jnp.bfloat16(1.0)
        zero = jnp.bfloat16(0.0)
        wcs = wc_ref[...][:, :m].T.astype(jnp.int16)
        iota = lax.broadcasted_iota(jnp.int16, (k, t), 0)

        def onehot(j):
            row = lax.broadcast_in_dim(wcs[j : j + 1, :], (k, t), (0, 1))
            return jnp.where(row == iota, one, zero)

        acc = jnp.zeros((d, t), jnp.float32)
        oh_t = onehot(0)
        for j in range(m):
            acc = acc + lax.dot_general(
                cbt_ref[j], oh_t, (((1,), (0,)), ((), ())),
                preferred_element_type=jnp.float32)
        out_ref[...] = acc.T

    return pl.pallas_call(
        body,
        grid=(grid,),
        in_specs=[
            pl.BlockSpec((t, w), lambda i: (i, 0)),
            pl.BlockSpec((m, d, k), lambda i: (0, 0, 0)),
        ],
        out_specs=pl.BlockSpec((t, d), lambda i: (i, 0)),
        out_shape=jax.ShapeDtypeStruct((n, d), jnp.float32),
        compiler_params=pltpu.CompilerParams(
            dimension_semantics=("arbitrary",)),
        interpret=interpret,
    )(wc, cbt)


def _combine_q8(wc, hi, lo, t=2048, interpret=False):
    n, m = wc.shape
    _, d, k = hi.shape
    grid = n // t

    def body(wc_ref, hi_ref, lo_ref, out_ref):
        one = jnp.int8(1)
        zero = jnp.int8(0)
        wcs = wc_ref[...].T.astype(jnp.uint8)
        iota = lax.broadcasted_iota(jnp.int16, (k, t), 0).astype(jnp.uint8)
        acc_h = jnp.zeros((d, t), jnp.int32)
        acc_l = jnp.zeros((d, t), jnp.int32)
        for j in range(m):
            row = lax.broadcast_in_dim(wcs[j : j + 1, :], (k, t), (0, 1))
            oh_t = jnp.where(row == iota, one, zero)
            acc_h = acc_h + lax.dot_general(
                hi_ref[j], oh_t, (((1,), (0,)), ((), ())),
                preferred_element_type=jnp.int32)
            acc_l = acc_l + lax.dot_general(
                lo_ref[j], oh_t, (((1,), (0,)), ((), ())),
                preferred_element_type=jnp.int32)
        out = acc_h.astype(jnp.float32) * (1.0 / 16.0) + \
            acc_l.astype(jnp.float32) * (1.0 / 4096.0)
        out_ref[...] = out.T

    return pl.pallas_call(
        body,
        grid=(grid,),
        in_specs=[
            pl.BlockSpec((t, m), lambda i: (i, 0)),
            pl.BlockSpec((m, d, k), lambda i: (0, 0, 0)),
            pl.BlockSpec((m, d, k), lambda i: (0, 0, 0)),
        ],
        out_specs=pl.BlockSpec((t, d), lambda i: (i, 0)),
        out_shape=jax.ShapeDtypeStruct((n, d), jnp.float32),
        compiler_params=pltpu.CompilerParams(
            dimension_semantics=("parallel",)),
        interpret=interpret,
    )(wc, hi, lo)


def _split_q8(codebook):
    q = jnp.round(codebook.transpose(0, 2, 1) * 4096.0)
    h = jnp.floor((q + 128.0) / 256.0)
    l = q - h * 256.0
    return h.astype(jnp.int8), l.astype(jnp.int8)


def kernel(x, codes, codebook):
    b, l = x.shape
    _, _, d = codebook.shape
    n = b * l
    codes_p = jnp.pad(codes, ((0, 0), (0, 128 - codes.shape[1])))
    wc = _gather_codes_tct(codes_p, x.reshape(n))
    out = _combine(wc, codebook.transpose(0, 2, 1).astype(jnp.bfloat16))
    return out.reshape(b, l, d)


# speedup vs baseline: 1.0014x; 1.0014x over previous
import jax
import jax.numpy as jnp
from jax import lax
from jax.experimental import pallas as pl
from jax.experimental.pallas import tpu as pltpu
from jax.experimental.pallas import tpu_sc as plsc


def _gather_codes(codes_p, idx):
    n = idx.shape[0]
    _, w = codes_p.shape
    info = plsc.get_sparse_core_info()
    nc, ns = info.num_cores, info.num_subcores
    nw = nc * ns
    n_per_w = n // nw
    ch = 800
    nch = n_per_w // ch

    mesh = plsc.VectorSubcoreMesh(core_axis_name="c", subcore_axis_name="s")

    def body(codes_hbm, idx_hbm, out_hbm, idx_v, rows_v, sem):
        wid = lax.axis_index("s") * nc + lax.axis_index("c")
        base = wid * n_per_w

        def step(i, carry):
            off = base + i * ch
            pltpu.sync_copy(idx_hbm.at[pl.ds(off, ch)], idx_v)
            pltpu.async_copy(codes_hbm.at[idx_v], rows_v, sem).wait()
            pltpu.sync_copy(rows_v, out_hbm.at[pl.ds(off, ch)])
            return carry

        lax.fori_loop(0, nch, step, 0)

    f = pl.kernel(
        body,
        mesh=mesh,
        out_type=jax.ShapeDtypeStruct((n, w), jnp.int32),
        scratch_types=[
            pltpu.VMEM((ch,), jnp.int32),
            pltpu.VMEM((ch, w), jnp.int32),
            pltpu.SemaphoreType.DMA,
        ],
        compiler_params=pltpu.CompilerParams(use_tc_tiling_on_sc=True),
    )
    return f(codes_p, idx)


def _combine(wc, cbt, m, t=2048, interpret=False):
    n, w = wc.shape
    d, mk = cbt.shape
    k = mk // m
    grid = n // t

    def body(wc_ref, cbt_ref, out_ref):
        one = jnp.bfloat16(1.0)
        zero = jnp.bfloat16(0.0)
        wcs = wc_ref[...][:, :m].T.astype(jnp.int16)
        iota = lax.broadcasted_iota(jnp.int16, (k, t), 0)

        def onehot(j):
            row = lax.broadcast_in_dim(wcs[j : j + 1, :], (k, t), (0, 1))
            return jnp.where(row == iota, one, zero)

        acc = jnp.zeros((d, t), jnp.float32)
        for j in range(m):
            acc = acc + lax.dot_general(
                cbt_ref[:, j * k : (j + 1) * k], onehot(j),
                (((1,), (0,)), ((), ())),
                preferred_element_type=jnp.float32)
        out_ref[...] = acc.T

    return pl.pallas_call(
        body,
        grid=(grid,),
        in_specs=[
            pl.BlockSpec((t, w), lambda i: (i, 0)),
            pl.BlockSpec((d, mk), lambda i: (0, 0)),
        ],
        out_specs=pl.BlockSpec((t, d), lambda i: (i, 0)),
        out_shape=jax.ShapeDtypeStruct((n, d), jnp.float32),
        compiler_params=pltpu.CompilerParams(
            dimension_semantics=("arbitrary",)),
        interpret=interpret,
    )(wc, cbt)


def kernel(x, codes, codebook):
    b, l = x.shape
    m, k, d = codebook.shape
    n = b * l
    codes_p = jnp.pad(codes, ((0, 0), (0, 128 - codes.shape[1])))
    wc = _gather_codes(codes_p, x.reshape(n))
    cbt = codebook.transpose(2, 0, 1).reshape(d, m * k).astype(jnp.bfloat16)
    out = _combine(wc, cbt, m)
    return out.reshape(b, l, d)
